# Initial kernel scaffold; baseline (speedup 1.0000x reference)
import jax
import jax.numpy as jnp
from jax.experimental import pallas as pl


def kernel(x, x_len, W2d, b2d, Wproj, bproj, W1d, b1d, Wfc, bfc):
    raise NotImplementedError("write your pallas kernel here")


# speedup vs baseline: 1.3079x; 1.3079x over previous
import functools

import jax
import jax.numpy as jnp
from jax import lax
from jax.experimental import pallas as pl
from jax.experimental.pallas import tpu as pltpu

B = 16
S = 2048
IN_DIM = 128
HID = 256
CH = 32
BETA = 1.0
DOWNSAMPLE = 4.0
EPS = 1e-4
MAX_EXTRA = 4
T = int(S // DOWNSAMPLE)

S_CHUNK = 512
C_CHUNK = 512


def _fused_kernel(xlen_ref, x_ref, a0_ref, a1_ref, a2_ref, b2dt_ref,
                  wproj_ref, bproj_ref, w1d_ref, b1d_ref, wfc_ref, bfc_ref,
                  out_ref, delay_ref, stats_ref, xp_ref):
    b = pl.program_id(0)
    xlen = xlen_ref[b]

    x = x_ref[0]
    zrow = jnp.zeros((1, IN_DIM), jnp.float32)
    xm = jnp.concatenate([zrow, x[:-1]], axis=0)
    xp1 = jnp.concatenate([x[1:], zrow], axis=0)

    for i in range(S // S_CHUNK):
        lo = i * S_CHUNK
        h = (jnp.dot(xm[lo:lo + S_CHUNK], a0_ref[...],
                     preferred_element_type=jnp.float32, precision=lax.Precision.HIGHEST)
             + jnp.dot(x[lo:lo + S_CHUNK], a1_ref[...],
                       preferred_element_type=jnp.float32, precision=lax.Precision.HIGHEST)
             + jnp.dot(xp1[lo:lo + S_CHUNK], a2_ref[...],
                       preferred_element_type=jnp.float32, precision=lax.Precision.HIGHEST)
             + b2dt_ref[...])
        h = jnp.maximum(h, 0.0)
        xp_ref[lo:lo + S_CHUNK, :] = (
            jnp.dot(h, wproj_ref[...], preferred_element_type=jnp.float32, precision=lax.Precision.HIGHEST)
            + bproj_ref[...])

    xpv = xp_ref[...]
    zh = jnp.zeros((1, HID), jnp.float32)
    xpm = jnp.concatenate([zh, xpv[:-1]], axis=0)
    xpp = jnp.concatenate([xpv[1:], zh], axis=0)
    o = (jnp.dot(xpm, w1d_ref[0], preferred_element_type=jnp.float32, precision=lax.Precision.HIGHEST)
         + jnp.dot(xpv, w1d_ref[1], preferred_element_type=jnp.float32, precision=lax.Precision.HIGHEST)
         + jnp.dot(xpp, w1d_ref[2], preferred_element_type=jnp.float32, precision=lax.Precision.HIGHEST)
         + b1d_ref[...])
    o = jnp.maximum(o, 0.0)
    probl = jnp.sum(o * wfc_ref[...], axis=1, keepdims=True) + bfc_ref[...]
    prob = jax.nn.sigmoid(probl)

    sidx = lax.broadcasted_iota(jnp.int32, (S, 1), 0)
    alpha0 = jnp.where(sidx < xlen, prob, 0.0)
    asum = jnp.sum(alpha0, keepdims=True)
    tgt = jnp.maximum(xlen // 4, 1)
    tgt_f = tgt.astype(jnp.float32)
    desired = BETA * tgt_f + EPS
    alpha = alpha0 * (desired / asum)

    ir = lax.broadcasted_iota(jnp.int32, (C_CHUNK, 1), 0)
    ic = lax.broadcasted_iota(jnp.int32, (1, C_CHUNK), 1)
    tril = (ir >= ic).astype(jnp.float32)
    c_parts = []
    off = jnp.zeros((1, 1), jnp.float32)
    for i in range(S // C_CHUNK):
        a_i = alpha[i * C_CHUNK:(i + 1) * C_CHUNK]
        c_parts.append(jnp.dot(tril, a_i, preferred_element_type=jnp.float32, precision=lax.Precision.HIGHEST)
                       + off)
        off = off + jnp.sum(a_i, keepdims=True)
    c = jnp.concatenate(c_parts, axis=0)

    r = jnp.minimum(jnp.floor(c), float(T))
    l = jnp.concatenate([jnp.zeros((1, 1), jnp.float32), r[:-1]], axis=0)
    n = r - l
    rw = jnp.where(n > 0, c - r, 0.0)
    extra = jnp.maximum(n - 1.0, 0.0)
    lw = alpha - rw - extra
    emax = jnp.minimum(extra, float(MAX_EXTRA))

    t_row = lax.broadcasted_iota(jnp.int32, (1, T), 1).astype(jnp.float32)
    d = t_row - l
    wT = (rw * (t_row == r)
          + lw * (t_row == l)
          + ((d >= 1.0) & (d <= emax)).astype(jnp.float32))

    out_ref[0] = lax.dot_general(
        wT, xpv, (((0,), (0,)), ((), ())),
        preferred_element_type=jnp.float32, precision=lax.Precision.HIGHEST)
    src = lax.broadcasted_iota(jnp.int32, (S, 1), 0).astype(jnp.float32) + 1.0
    delay_ref[0, 0, :] = jnp.sum(wT * src, axis=0)

    lidx = lax.broadcasted_iota(jnp.int32, (1, 128), 1)
    stats_ref[0] = jnp.where(
        lidx == 0, asum,
        jnp.where(lidx == 1, tgt_f, 0.0))


def _build_banded(W2d):
    fp = jnp.arange(IN_DIM)[:, None]
    f = jnp.arange(IN_DIM)[None, :]
    mats = []
    for i in range(3):
        Ai = jnp.zeros((IN_DIM, IN_DIM, CH), jnp.float32)
        for j in range(3):
            Ej = (fp == f + j - 1).astype(jnp.float32)
            Ai = Ai + Ej[:, :, None] * W2d[:, 0, i, j][None, None, :]
        mats.append(Ai.reshape(IN_DIM, IN_DIM * CH))
    return mats


@jax.jit
def kernel(x, x_len, W2d, b2d, Wproj, bproj, W1d, b1d, Wfc, bfc):
    A0, A1, A2 = _build_banded(W2d)
    b2d_t = jnp.tile(b2d, IN_DIM)[None, :]
    w1d_t = jnp.transpose(W1d, (2, 1, 0))
    wfc_row = Wfc[:, 0][None, :]
    bfc_sq = bfc[None, :]
    bproj_r = bproj[None, :]
    b1d_r = b1d[None, :]

    full = lambda shp: pl.BlockSpec(shp, lambda b, xl: (0,) * len(shp))
    out, delay, stats = pl.pallas_call(
        _fused_kernel,
        grid_spec=pltpu.PrefetchScalarGridSpec(
            num_scalar_prefetch=1,
            grid=(B,),
            in_specs=[
                pl.BlockSpec((1, S, IN_DIM), lambda b, xl: (b, 0, 0)),
                full((IN_DIM, IN_DIM * CH)),
                full((IN_DIM, IN_DIM * CH)),
                full((IN_DIM, IN_DIM * CH)),
                full((1, IN_DIM * CH)),
                full((IN_DIM * CH, HID)),
                full((1, HID)),
                full((3, HID, HID)),
                full((1, HID)),
                full((1, HID)),
                full((1, 1)),
            ],
            out_specs=[
                pl.BlockSpec((1, T, HID), lambda b, xl: (b, 0, 0)),
                pl.BlockSpec((1, 1, T), lambda b, xl: (b, 0, 0)),
                pl.BlockSpec((1, 1, 128), lambda b, xl: (b, 0, 0)),
            ],
            scratch_shapes=[pltpu.VMEM((S, HID), jnp.float32)],
        ),
        out_shape=[
            jax.ShapeDtypeStruct((B, T, HID), jnp.float32),
            jax.ShapeDtypeStruct((B, 1, T), jnp.float32),
            jax.ShapeDtypeStruct((B, 1, 128), jnp.float32),
        ],
    )(x_len, x, A0, A1, A2, b2d_t, Wproj, bproj_r, w1d_t, b1d_r,
      wfc_row, bfc_sq)

    alpha_sum = stats[:, 0, 0]
    tgt_len = stats[:, 0, 1].astype(jnp.int32)
    return out, tgt_len, alpha_sum, delay[:, 0, :]


# speedup vs baseline: 2.3107x; 1.7667x over previous
import jax
import jax.numpy as jnp
from jax import lax
from jax.experimental import pallas as pl
from jax.experimental.pallas import tpu as pltpu

B = 16
S = 2048
IN_DIM = 128
HID = 256
CH = 32
BETA = 1.0
DOWNSAMPLE = 4.0
EPS = 1e-4
MAX_EXTRA = 4
T = int(S // DOWNSAMPLE)

C_CHUNK = 512

_HI = dict(preferred_element_type=jnp.float32, precision=lax.Precision.HIGHEST)


def _fused_kernel(xlen_ref, scal_ref, x_ref, wproj_ref, bproj_ref,
                  w1d_ref, b1d_ref, wfc_ref, bfc_ref,
                  out_ref, delay_ref, stats_ref, xp_ref):
    b = pl.program_id(0)
    xlen = xlen_ref[b]

    x = x_ref[0]
    zlane = jnp.zeros((S, 1), jnp.float32)
    zrow = jnp.zeros((1, IN_DIM), jnp.float32)
    zm = jnp.concatenate([zlane, x[:, :-1]], axis=1)
    zp = jnp.concatenate([x[:, 1:], zlane], axis=1)
    taps = []
    for z in (zm, x, zp):
        pass
    for i in range(3):
        for j in range(3):
            z = (zm, x, zp)[j]
            if i == 0:
                t = jnp.concatenate([zrow, z[:-1]], axis=0)
            elif i == 1:
                t = z
            else:
                t = jnp.concatenate([z[1:], zrow], axis=0)
            taps.append(t)

    def ch_body(ch, xp_acc):
        acc = taps[0] * scal_ref[ch * 9]
        for k in range(1, 9):
            acc = acc + taps[k] * scal_ref[ch * 9 + k]
        hch = jnp.maximum(acc + scal_ref[CH * 9 + ch], 0.0)
        return xp_acc + jnp.dot(hch, wproj_ref[ch], **_HI)

    xp_acc = lax.fori_loop(0, CH, ch_body,
                           jnp.zeros((S, HID), jnp.float32))
    xp_ref[...] = xp_acc + bproj_ref[...]

    xpv = xp_ref[...]
    zh = jnp.zeros((1, HID), jnp.float32)
    xpm = jnp.concatenate([zh, xpv[:-1]], axis=0)
    xpp = jnp.concatenate([xpv[1:], zh], axis=0)
    o = (jnp.dot(xpm, w1d_ref[0], **_HI)
         + jnp.dot(xpv, w1d_ref[1], **_HI)
         + jnp.dot(xpp, w1d_ref[2], **_HI)
         + b1d_ref[...])
    o = jnp.maximum(o, 0.0)
    probl = jnp.sum(o * wfc_ref[...], axis=1, keepdims=True) + bfc_ref[...]
    prob = jax.nn.sigmoid(probl)

    sidx = lax.broadcasted_iota(jnp.int32, (S, 1), 0)
    alpha0 = jnp.where(sidx < xlen, prob, 0.0)
    asum = jnp.sum(alpha0, keepdims=True)
    tgt = jnp.maximum(xlen // 4, 1)
    tgt_f = tgt.astype(jnp.float32)
    desired = BETA * tgt_f + EPS
    alpha = alpha0 * (desired / asum)

    ir = lax.broadcasted_iota(jnp.int32, (C_CHUNK, 1), 0)
    ic = lax.broadcasted_iota(jnp.int32, (1, C_CHUNK), 1)
    tril = (ir >= ic).astype(jnp.float32)
    c_parts = []
    off = jnp.zeros((1, 1), jnp.float32)
    for i in range(S // C_CHUNK):
        a_i = alpha[i * C_CHUNK:(i + 1) * C_CHUNK]
        c_parts.append(jnp.dot(tril, a_i, **_HI) + off)
        off = off + jnp.sum(a_i, keepdims=True)
    c = jnp.concatenate(c_parts, axis=0)

    r = jnp.minimum(jnp.floor(c), float(T))
    l = jnp.concatenate([jnp.zeros((1, 1), jnp.float32), r[:-1]], axis=0)
    n = r - l
    rw = jnp.where(n > 0, c - r, 0.0)
    extra = jnp.maximum(n - 1.0, 0.0)
    lw = alpha - rw - extra
    emax = jnp.minimum(extra, float(MAX_EXTRA))

    t_row = lax.broadcasted_iota(jnp.int32, (1, T), 1).astype(jnp.float32)
    d = t_row - l
    wT = (rw * (t_row == r)
          + lw * (t_row == l)
          + ((d >= 1.0) & (d <= emax)).astype(jnp.float32))

    out_ref[0] = lax.dot_general(
        wT, xpv, (((0,), (0,)), ((), ())), **_HI)
    src = lax.broadcasted_iota(jnp.int32, (S, 1), 0).astype(jnp.float32) + 1.0
    delay_ref[0, 0, :] = jnp.sum(wT * src, axis=0)

    lidx = lax.broadcasted_iota(jnp.int32, (1, 128), 1)
    stats_ref[0] = jnp.where(
        lidx == 0, asum,
        jnp.where(lidx == 1, tgt_f, 0.0))


@jax.jit
def kernel(x, x_len, W2d, b2d, Wproj, bproj, W1d, b1d, Wfc, bfc):
    scal = jnp.concatenate([W2d.reshape(-1), b2d])
    wproj_p = Wproj.reshape(IN_DIM, CH, HID).transpose(1, 0, 2)
    w1d_t = jnp.transpose(W1d, (2, 1, 0))
    wfc_row = Wfc[:, 0][None, :]
    bfc_sq = bfc[None, :]
    bproj_r = bproj[None, :]
    b1d_r = b1d[None, :]

    full = lambda shp: pl.BlockSpec(shp, lambda b, *_: (0,) * len(shp))
    out, delay, stats = pl.pallas_call(
        _fused_kernel,
        grid_spec=pltpu.PrefetchScalarGridSpec(
            num_scalar_prefetch=2,
            grid=(B,),
            in_specs=[
                pl.BlockSpec((1, S, IN_DIM), lambda b, *_: (b, 0, 0)),
                full((CH, IN_DIM, HID)),
                full((1, HID)),
                full((3, HID, HID)),
                full((1, HID)),
                full((1, HID)),
                full((1, 1)),
            ],
            out_specs=[
                pl.BlockSpec((1, T, HID), lambda b, *_: (b, 0, 0)),
                pl.BlockSpec((1, 1, T), lambda b, *_: (b, 0, 0)),
                pl.BlockSpec((1, 1, 128), lambda b, *_: (b, 0, 0)),
            ],
            scratch_shapes=[pltpu.VMEM((S, HID), jnp.float32)],
        ),
        out_shape=[
            jax.ShapeDtypeStruct((B, T, HID), jnp.float32),
            jax.ShapeDtypeStruct((B, 1, T), jnp.float32),
            jax.ShapeDtypeStruct((B, 1, 128), jnp.float32),
        ],
    )(x_len, scal, x, wproj_p, bproj_r, w1d_t, b1d_r, wfc_row, bfc_sq)

    alpha_sum = stats[:, 0, 0]
    tgt_len = stats[:, 0, 1].astype(jnp.int32)
    return out, tgt_len, alpha_sum, delay[:, 0, :]


# speedup vs baseline: 3.5029x; 1.5160x over previous
import jax
import jax.numpy as jnp
from jax import lax
from jax.experimental import pallas as pl
from jax.experimental.pallas import tpu as pltpu

B = 16
S = 2048
IN_DIM = 128
HID = 256
CH = 32
BETA = 1.0
DOWNSAMPLE = 4.0
EPS = 1e-4
MAX_EXTRA = 4
T = int(S // DOWNSAMPLE)

C_CHUNK = 512

_HI = dict(preferred_element_type=jnp.float32, precision=lax.Precision.HIGHEST)
_DF = dict(preferred_element_type=jnp.float32)


def _split_bf16(v):
    hi = v.astype(jnp.bfloat16)
    lo = (v - hi.astype(jnp.float32)).astype(jnp.bfloat16)
    return hi, lo


def _dot3(a, b_hi, b_lo):
    a_hi, a_lo = _split_bf16(a)
    return (jnp.dot(a_hi, b_hi, **_DF)
            + jnp.dot(a_hi, b_lo, **_DF)
            + jnp.dot(a_lo, b_hi, **_DF))


def _fused_kernel(xlen_ref, scal_ref, x_ref, wproj_hi_ref, wproj_lo_ref,
                  bproj_ref, w1d_hi_ref, w1d_lo_ref, b1d_ref, wfc_ref,
                  bfc_ref, out_ref, delay_ref, stats_ref, xp_ref):
    b = pl.program_id(0)
    xlen = xlen_ref[b]

    x = x_ref[0]
    zlane = jnp.zeros((S, 1), jnp.float32)
    zrow = jnp.zeros((1, IN_DIM), jnp.float32)
    zm = jnp.concatenate([zlane, x[:, :-1]], axis=1)
    zp = jnp.concatenate([x[:, 1:], zlane], axis=1)
    taps = []
    for z in (zm, x, zp):
        pass
    for i in range(3):
        for j in range(3):
            z = (zm, x, zp)[j]
            if i == 0:
                t = jnp.concatenate([zrow, z[:-1]], axis=0)
            elif i == 1:
                t = z
            else:
                t = jnp.concatenate([z[1:], zrow], axis=0)
            taps.append(t)

    def ch_body(ch, xp_acc):
        acc = taps[0] * scal_ref[ch * 9]
        for k in range(1, 9):
            acc = acc + taps[k] * scal_ref[ch * 9 + k]
        hch = jnp.maximum(acc + scal_ref[CH * 9 + ch], 0.0)
        return xp_acc + _dot3(hch, wproj_hi_ref[ch], wproj_lo_ref[ch])

    xp_acc = lax.fori_loop(0, CH, ch_body,
                           jnp.zeros((S, HID), jnp.float32))
    xp_ref[...] = xp_acc + bproj_ref[...]

    xpv = xp_ref[...]
    zh = jnp.zeros((1, HID), jnp.float32)
    xpm = jnp.concatenate([zh, xpv[:-1]], axis=0)
    xpp = jnp.concatenate([xpv[1:], zh], axis=0)
    o = (_dot3(xpm, w1d_hi_ref[0], w1d_lo_ref[0])
         + _dot3(xpv, w1d_hi_ref[1], w1d_lo_ref[1])
         + _dot3(xpp, w1d_hi_ref[2], w1d_lo_ref[2])
         + b1d_ref[...])
    o = jnp.maximum(o, 0.0)
    probl = jnp.sum(o * wfc_ref[...], axis=1, keepdims=True) + bfc_ref[...]
    prob = jax.nn.sigmoid(probl)

    sidx = lax.broadcasted_iota(jnp.int32, (S, 1), 0)
    alpha0 = jnp.where(sidx < xlen, prob, 0.0)
    asum = jnp.sum(alpha0, keepdims=True)
    tgt = jnp.maximum(xlen // 4, 1)
    tgt_f = tgt.astype(jnp.float32)
    desired = BETA * tgt_f + EPS
    alpha = alpha0 * (desired / asum)

    ir = lax.broadcasted_iota(jnp.int32, (C_CHUNK, 1), 0)
    ic = lax.broadcasted_iota(jnp.int32, (1, C_CHUNK), 1)
    tril = (ir >= ic).astype(jnp.float32)
    c_parts = []
    off = jnp.zeros((1, 1), jnp.float32)
    for i in range(S // C_CHUNK):
        a_i = alpha[i * C_CHUNK:(i + 1) * C_CHUNK]
        c_parts.append(jnp.dot(tril, a_i, **_HI) + off)
        off = off + jnp.sum(a_i, keepdims=True)
    c = jnp.concatenate(c_parts, axis=0)

    r = jnp.minimum(jnp.floor(c), float(T))
    l = jnp.concatenate([jnp.zeros((1, 1), jnp.float32), r[:-1]], axis=0)
    n = r - l
    rw = jnp.where(n > 0, c - r, 0.0)
    extra = jnp.maximum(n - 1.0, 0.0)
    lw = alpha - rw - extra
    emax = jnp.minimum(extra, float(MAX_EXTRA))

    t_row = lax.broadcasted_iota(jnp.int32, (1, T), 1).astype(jnp.float32)
    d = t_row - l
    wT = (rw * (t_row == r)
          + lw * (t_row == l)
          + ((d >= 1.0) & (d <= emax)).astype(jnp.float32))

    out_ref[0] = lax.dot_general(
        wT, xpv, (((0,), (0,)), ((), ())), **_HI)
    src = lax.broadcasted_iota(jnp.int32, (S, 1), 0).astype(jnp.float32) + 1.0
    delay_ref[0, 0, :] = jnp.sum(wT * src, axis=0)

    lidx = lax.broadcasted_iota(jnp.int32, (1, 128), 1)
    stats_ref[0] = jnp.where(
        lidx == 0, asum,
        jnp.where(lidx == 1, tgt_f, 0.0))


@jax.jit
def kernel(x, x_len, W2d, b2d, Wproj, bproj, W1d, b1d, Wfc, bfc):
    scal = jnp.concatenate([W2d.reshape(-1), b2d])
    wproj_p = Wproj.reshape(IN_DIM, CH, HID).transpose(1, 0, 2)
    wproj_hi = wproj_p.astype(jnp.bfloat16)
    wproj_lo = (wproj_p - wproj_hi.astype(jnp.float32)).astype(jnp.bfloat16)
    w1d_t = jnp.transpose(W1d, (2, 1, 0))
    w1d_hi = w1d_t.astype(jnp.bfloat16)
    w1d_lo = (w1d_t - w1d_hi.astype(jnp.float32)).astype(jnp.bfloat16)
    wfc_row = Wfc[:, 0][None, :]
    bfc_sq = bfc[None, :]
    bproj_r = bproj[None, :]
    b1d_r = b1d[None, :]

    full = lambda shp: pl.BlockSpec(shp, lambda b, *_: (0,) * len(shp))
    out, delay, stats = pl.pallas_call(
        _fused_kernel,
        grid_spec=pltpu.PrefetchScalarGridSpec(
            num_scalar_prefetch=2,
            grid=(B,),
            in_specs=[
                pl.BlockSpec((1, S, IN_DIM), lambda b, *_: (b, 0, 0)),
                full((CH, IN_DIM, HID)),
                full((CH, IN_DIM, HID)),
                full((1, HID)),
                full((3, HID, HID)),
                full((3, HID, HID)),
                full((1, HID)),
                full((1, HID)),
                full((1, 1)),
            ],
            out_specs=[
                pl.BlockSpec((1, T, HID), lambda b, *_: (b, 0, 0)),
                pl.BlockSpec((1, 1, T), lambda b, *_: (b, 0, 0)),
                pl.BlockSpec((1, 1, 128), lambda b, *_: (b, 0, 0)),
            ],
            scratch_shapes=[pltpu.VMEM((S, HID), jnp.float32)],
        ),
        out_shape=[
            jax.ShapeDtypeStruct((B, T, HID), jnp.float32),
            jax.ShapeDtypeStruct((B, 1, T), jnp.float32),
            jax.ShapeDtypeStruct((B, 1, 128), jnp.float32),
        ],
    )(x_len, scal, x, wproj_hi, wproj_lo, bproj_r, w1d_hi, w1d_lo,
      b1d_r, wfc_row, bfc_sq)

    alpha_sum = stats[:, 0, 0]
    tgt_len = stats[:, 0, 1].astype(jnp.int32)
    return out, tgt_len, alpha_sum, delay[:, 0, :]


# speedup vs baseline: 3.6006x; 1.0279x over previous
import jax
import jax.numpy as jnp
from jax import lax
from jax.experimental import pallas as pl
from jax.experimental.pallas import tpu as pltpu

B = 16
S = 2048
IN_DIM = 128
HID = 256
CH = 32
BETA = 1.0
DOWNSAMPLE = 4.0
EPS = 1e-4
MAX_EXTRA = 4
T = int(S // DOWNSAMPLE)

C_CHUNK = 512

_HI = dict(preferred_element_type=jnp.float32, precision=lax.Precision.HIGHEST)
_DF = dict(preferred_element_type=jnp.float32)


def _split_bf16(v):
    hi = v.astype(jnp.bfloat16)
    lo = (v - hi.astype(jnp.float32)).astype(jnp.bfloat16)
    return hi, lo


def _dot3(a, b_hi, b_lo):
    a_hi, a_lo = _split_bf16(a)
    return (jnp.dot(a_hi, b_hi, **_DF)
            + jnp.dot(a_hi, b_lo, **_DF)
            + jnp.dot(a_lo, b_hi, **_DF))


def _fused_kernel(xlen_ref, scal_ref, x_ref, wproj_hi_ref, wproj_lo_ref,
                  bproj_ref, w1d_hi_ref, w1d_lo_ref, b1d_ref, wfc_ref,
                  bfc_ref, out_ref, delay_ref, stats_ref, xp_ref):
    b = pl.program_id(0)
    xlen = xlen_ref[b]

    x = x_ref[0]
    zlane = jnp.zeros((S, 1), jnp.float32)
    zrow = jnp.zeros((1, IN_DIM), jnp.float32)
    zm = jnp.concatenate([zlane, x[:, :-1]], axis=1)
    zp = jnp.concatenate([x[:, 1:], zlane], axis=1)
    taps = []
    for z in (zm, x, zp):
        pass
    for i in range(3):
        for j in range(3):
            z = (zm, x, zp)[j]
            if i == 0:
                t = jnp.concatenate([zrow, z[:-1]], axis=0)
            elif i == 1:
                t = z
            else:
                t = jnp.concatenate([z[1:], zrow], axis=0)
            taps.append(t)

    def ch_body(ch, xp_acc):
        acc = taps[0] * scal_ref[ch * 9]
        for k in range(1, 9):
            acc = acc + taps[k] * scal_ref[ch * 9 + k]
        hch = jnp.maximum(acc + scal_ref[CH * 9 + ch], 0.0)
        return xp_acc + _dot3(hch, wproj_hi_ref[ch], wproj_lo_ref[ch])

    xp_acc = lax.fori_loop(0, CH, ch_body,
                           jnp.zeros((S, HID), jnp.float32))
    xp_ref[...] = xp_acc + bproj_ref[...]

    xpv = xp_ref[...]
    zh = jnp.zeros((1, HID), jnp.float32)
    xpm = jnp.concatenate([zh, xpv[:-1]], axis=0)
    xpp = jnp.concatenate([xpv[1:], zh], axis=0)
    o = (_dot3(xpm, w1d_hi_ref[0], w1d_lo_ref[0])
         + _dot3(xpv, w1d_hi_ref[1], w1d_lo_ref[1])
         + _dot3(xpp, w1d_hi_ref[2], w1d_lo_ref[2])
         + b1d_ref[...])
    o = jnp.maximum(o, 0.0)
    probl = jnp.sum(o * wfc_ref[...], axis=1, keepdims=True) + bfc_ref[...]
    prob = jax.nn.sigmoid(probl)

    sidx = lax.broadcasted_iota(jnp.int32, (S, 1), 0)
    alpha0 = jnp.where(sidx < xlen, prob, 0.0)
    asum = jnp.sum(alpha0, keepdims=True)
    tgt = jnp.maximum(xlen // 4, 1)
    tgt_f = tgt.astype(jnp.float32)
    desired = BETA * tgt_f + EPS
    alpha = alpha0 * (desired / asum)

    ir = lax.broadcasted_iota(jnp.int32, (C_CHUNK, 1), 0)
    ic = lax.broadcasted_iota(jnp.int32, (1, C_CHUNK), 1)
    tril = (ir >= ic).astype(jnp.float32)
    c_parts = []
    off = jnp.zeros((1, 1), jnp.float32)
    for i in range(S // C_CHUNK):
        a_i = alpha[i * C_CHUNK:(i + 1) * C_CHUNK]
        c_parts.append(jnp.dot(tril, a_i, **_HI) + off)
        off = off + jnp.sum(a_i, keepdims=True)
    c = jnp.concatenate(c_parts, axis=0)

    r = jnp.minimum(jnp.floor(c), float(T))
    l = jnp.concatenate([jnp.zeros((1, 1), jnp.float32), r[:-1]], axis=0)
    n = r - l
    rw = jnp.where(n > 0, c - r, 0.0)
    extra = jnp.maximum(n - 1.0, 0.0)
    lw = alpha - rw - extra
    emax = jnp.minimum(extra, float(MAX_EXTRA))

    t_row = lax.broadcasted_iota(jnp.int32, (1, T), 1).astype(jnp.float32)
    d = t_row - l
    wT = (rw * (t_row == r)
          + lw * (t_row == l)
          + ((d >= 1.0) & (d <= emax)).astype(jnp.float32))

    wt_hi, wt_lo = _split_bf16(wT)
    xp_hi, xp_lo = _split_bf16(xpv)
    dnum = (((0,), (0,)), ((), ()))
    out_ref[0] = (lax.dot_general(wt_hi, xp_hi, dnum, **_DF)
                  + lax.dot_general(wt_hi, xp_lo, dnum, **_DF)
                  + lax.dot_general(wt_lo, xp_hi, dnum, **_DF))
    src = lax.broadcasted_iota(jnp.int32, (S, 1), 0).astype(jnp.float32) + 1.0
    delay_ref[0, 0, :] = jnp.sum(wT * src, axis=0)

    lidx = lax.broadcasted_iota(jnp.int32, (1, 128), 1)
    stats_ref[0] = jnp.where(
        lidx == 0, asum,
        jnp.where(lidx == 1, tgt_f, 0.0))


@jax.jit
def kernel(x, x_len, W2d, b2d, Wproj, bproj, W1d, b1d, Wfc, bfc):
    scal = jnp.concatenate([W2d.reshape(-1), b2d])
    wproj_p = Wproj.reshape(IN_DIM, CH, HID).transpose(1, 0, 2)
    wproj_hi = wproj_p.astype(jnp.bfloat16)
    wproj_lo = (wproj_p - wproj_hi.astype(jnp.float32)).astype(jnp.bfloat16)
    w1d_t = jnp.transpose(W1d, (2, 1, 0))
    w1d_hi = w1d_t.astype(jnp.bfloat16)
    w1d_lo = (w1d_t - w1d_hi.astype(jnp.float32)).astype(jnp.bfloat16)
    wfc_row = Wfc[:, 0][None, :]
    bfc_sq = bfc[None, :]
    bproj_r = bproj[None, :]
    b1d_r = b1d[None, :]

    full = lambda shp: pl.BlockSpec(shp, lambda b, *_: (0,) * len(shp))
    out, delay, stats = pl.pallas_call(
        _fused_kernel,
        grid_spec=pltpu.PrefetchScalarGridSpec(
            num_scalar_prefetch=2,
            grid=(B,),
            in_specs=[
                pl.BlockSpec((1, S, IN_DIM), lambda b, *_: (b, 0, 0)),
                full((CH, IN_DIM, HID)),
                full((CH, IN_DIM, HID)),
                full((1, HID)),
                full((3, HID, HID)),
                full((3, HID, HID)),
                full((1, HID)),
                full((1, HID)),
                full((1, 1)),
            ],
            out_specs=[
                pl.BlockSpec((1, T, HID), lambda b, *_: (b, 0, 0)),
                pl.BlockSpec((1, 1, T), lambda b, *_: (b, 0, 0)),
                pl.BlockSpec((1, 1, 128), lambda b, *_: (b, 0, 0)),
            ],
            scratch_shapes=[pltpu.VMEM((S, HID), jnp.float32)],
        ),
        out_shape=[
            jax.ShapeDtypeStruct((B, T, HID), jnp.float32),
            jax.ShapeDtypeStruct((B, 1, T), jnp.float32),
            jax.ShapeDtypeStruct((B, 1, 128), jnp.float32),
        ],
    )(x_len, scal, x, wproj_hi, wproj_lo, bproj_r, w1d_hi, w1d_lo,
      b1d_r, wfc_row, bfc_sq)

    alpha_sum = stats[:, 0, 0]
    tgt_len = stats[:, 0, 1].astype(jnp.int32)
    return out, tgt_len, alpha_sum, delay[:, 0, :]
